# Initial kernel scaffold; baseline (speedup 1.0000x reference)
import functools

import jax
import jax.numpy as jnp
from jax import lax
from jax.experimental import pallas as pl
from jax.experimental.pallas import tpu as pltpu
from jax.experimental.pallas import tpu_sc as plsc

N = 10000
D = 128
E = 320000

NC = 2
NS = 16
NW = NC * NS

NP = 10240
EPT = 10240
EP = EPT * NW
C = 128
NCH = EPT // C
RPS = NP // NS

_mesh = plsc.VectorSubcoreMesh(core_axis_name="c", subcore_axis_name="s")


@functools.partial(
    pl.kernel,
    out_type=jax.ShapeDtypeStruct((3, NC, NP, 8), jnp.float32),
    mesh=_mesh,
    scratch_types=[
        pltpu.VMEM_SHARED((NP, 8), jnp.float32),
        pltpu.VMEM((C, 8), jnp.float32),
        pltpu.VMEM((C, 8), jnp.float32),
        pltpu.VMEM((NCH, C), jnp.int32),
    ],
)
def _deg_kernel(dst_r, ones8, zeros8, out, acc, ones_v, zer_v, didx_v):
    c = lax.axis_index("c")
    s = lax.axis_index("s")
    w = c * NS + s
    pltpu.sync_copy(ones8, ones_v)
    pltpu.sync_copy(zeros8, zer_v)
    for k in range(3):
        for r in range(RPS // C):
            pltpu.sync_copy(zer_v, acc.at[pl.ds(s * RPS + r * C, C)])
        plsc.subcore_barrier()
        pltpu.sync_copy(dst_r.at[k, w], didx_v)

        def body(j, carry):
            pltpu.sync_copy(ones_v, acc.at[didx_v.at[j]], add=True)
            return carry

        lax.fori_loop(0, NCH, body, 0)
        plsc.subcore_barrier()
        pltpu.sync_copy(
            acc.at[pl.ds(s * RPS, RPS)], out.at[k, c, pl.ds(s * RPS, RPS)]
        )
        plsc.subcore_barrier()


@functools.partial(
    pl.kernel,
    out_type=jax.ShapeDtypeStruct((3, NC, NP, D), jnp.float32),
    mesh=_mesh,
    scratch_types=[
        pltpu.VMEM_SHARED((NP, D), jnp.float32),
        pltpu.VMEM((C, D), jnp.float32),
        pltpu.VMEM((C, D), jnp.float32),
        pltpu.VMEM((NCH, C), jnp.int32),
        pltpu.VMEM((NCH, C), jnp.int32),
        pltpu.SemaphoreType.DMA,
    ],
)
def _spmm_kernel(h0, h1, h2, src_r, dst_r, zeros, out,
                 acc, rbuf, zer_v, sidx_v, didx_v, sem):
    c = lax.axis_index("c")
    s = lax.axis_index("s")
    w = c * NS + s
    pltpu.sync_copy(zeros, zer_v)
    for k, h in enumerate((h0, h1, h2)):
        for r in range(RPS // C):
            pltpu.sync_copy(zer_v, acc.at[pl.ds(s * RPS + r * C, C)])
        plsc.subcore_barrier()
        pltpu.sync_copy(src_r.at[k, w], sidx_v)
        pltpu.sync_copy(dst_r.at[k, w], didx_v)

        def body(j, carry):
            pltpu.async_copy(h.at[sidx_v.at[j]], rbuf, sem).wait()
            pltpu.sync_copy(rbuf, acc.at[didx_v.at[j]], add=True)
            return carry

        lax.fori_loop(0, NCH, body, 0)
        plsc.subcore_barrier()
        pltpu.sync_copy(
            acc.at[pl.ds(s * RPS, RPS)], out.at[k, c, pl.ds(s * RPS, RPS)]
        )
        plsc.subcore_barrier()


_BLK = 1024
_GR = NP // _BLK


def _dot(a, b):
    return jnp.dot(a, b, preferred_element_type=jnp.float32,
                   precision=lax.Precision.HIGHEST)


def _prep_body(x_ref, w1_ref, degp_ref, out_ref):
    deg = degp_ref[0, 0] + degp_ref[0, 1]
    dinv = lax.rsqrt(deg[:, 0:1] + 1.0)
    out_ref[0] = dinv * _dot(x_ref[...], w1_ref[0])


def _prep(x_pad, w1s, degp):
    return pl.pallas_call(
        _prep_body,
        grid=(3, _GR),
        in_specs=[
            pl.BlockSpec((_BLK, D), lambda k, i: (i, 0)),
            pl.BlockSpec((1, D, D), lambda k, i: (k, 0, 0)),
            pl.BlockSpec((1, NC, _BLK, 8), lambda k, i: (k, 0, i, 0)),
        ],
        out_specs=pl.BlockSpec((1, _BLK, D), lambda k, i: (k, i, 0)),
        out_shape=jax.ShapeDtypeStruct((3, NP, D), jnp.float32),
    )(x_pad, w1s, degp)


def _mid_body(p_ref, ys_ref, degp_ref, w2_ref, b1_ref, g1_ref, be1_ref,
              a1_ref, out_ref):
    deg = degp_ref[0, 0] + degp_ref[0, 1]
    dinv = lax.rsqrt(deg[:, 0:1] + 1.0)
    h = dinv * (p_ref[0, 0] + p_ref[0, 1] + ys_ref[0]) + b1_ref[0]
    m = jnp.mean(h, axis=-1, keepdims=True)
    v = jnp.mean(jnp.square(h - m), axis=-1, keepdims=True)
    hn = (h - m) * lax.rsqrt(v + 1e-5) * g1_ref[0] + be1_ref[0]
    hp = jnp.where(hn >= 0, hn, a1_ref[0] * hn)
    out_ref[0] = dinv * _dot(hp, w2_ref[0])


def _mid(p, ys, degp, w2s, b1s, g1s, be1s, a1s):
    vec = pl.BlockSpec((1, D), lambda k, i: (k, 0))
    return pl.pallas_call(
        _mid_body,
        grid=(3, _GR),
        in_specs=[
            pl.BlockSpec((1, NC, _BLK, D), lambda k, i: (k, 0, i, 0)),
            pl.BlockSpec((1, _BLK, D), lambda k, i: (k, i, 0)),
            pl.BlockSpec((1, NC, _BLK, 8), lambda k, i: (k, 0, i, 0)),
            pl.BlockSpec((1, D, D), lambda k, i: (k, 0, 0)),
            vec, vec, vec, vec,
        ],
        out_specs=pl.BlockSpec((1, _BLK, D), lambda k, i: (k, i, 0)),
        out_shape=jax.ShapeDtypeStruct((3, NP, D), jnp.float32),
    )(p, ys, degp, w2s, b1s, g1s, be1s, a1s)


def _fin_body(q_ref, zs_ref, degp_ref, b2_ref, wm_ref, bm_ref, out_ref):
    acc = jnp.zeros((_BLK, D), jnp.float32)
    for k in range(3):
        deg = degp_ref[k, 0] + degp_ref[k, 1]
        dinv = lax.rsqrt(deg[:, 0:1] + 1.0)
        hk = dinv * (q_ref[k, 0] + q_ref[k, 1] + zs_ref[k]) + b2_ref[k]
        acc = acc + _dot(hk, wm_ref[k]) + hk
    out_ref[...] = jnp.maximum(acc + bm_ref[0], 0.0)


def _fin(q, zs, degp, b2s, wms, bm):
    return pl.pallas_call(
        _fin_body,
        grid=(_GR,),
        in_specs=[
            pl.BlockSpec((3, NC, _BLK, D), lambda i: (0, 0, i, 0)),
            pl.BlockSpec((3, _BLK, D), lambda i: (0, i, 0)),
            pl.BlockSpec((3, NC, _BLK, 8), lambda i: (0, 0, i, 0)),
            pl.BlockSpec((3, D), lambda i: (0, 0)),
            pl.BlockSpec((3, D, D), lambda i: (0, 0, 0)),
            pl.BlockSpec((1, D), lambda i: (0, 0)),
        ],
        out_specs=pl.BlockSpec((_BLK, D), lambda i: (i, 0)),
        out_shape=jax.ShapeDtypeStruct((NP, D), jnp.float32),
    )(q, zs, degp, b2s, wms, bm)


def _prep_edges(adj):
    pad = jnp.full((EP - E,), NP - 1, dtype=jnp.int32)
    src = jnp.concatenate([adj[0].astype(jnp.int32), pad]).reshape(NW, NCH, C)
    dst = jnp.concatenate([adj[1].astype(jnp.int32), pad]).reshape(NW, NCH, C)
    return src, dst


@jax.jit
def kernel(x, params, adj_intra, adj_masked, adj):
    x_pad = jnp.pad(x, ((0, NP - N), (0, 0)))
    edges = [_prep_edges(a) for a in (adj_intra, adj_masked, adj)]
    src_r = jnp.stack([e[0] for e in edges])
    dst_r = jnp.stack([e[1] for e in edges])

    encs = [params["intra"], params["masked"], params["full"]]
    w1s = jnp.stack([p["W1"] for p in encs])
    w2s = jnp.stack([p["W2"] for p in encs])
    b1s = jnp.stack([p["b1"] for p in encs])
    b2s = jnp.stack([p["b2"] for p in encs])
    g1s = jnp.stack([p["g1"] for p in encs])
    be1s = jnp.stack([p["be1"] for p in encs])
    a1s = jnp.stack([jnp.full((D,), p["a1"], jnp.float32) for p in encs])
    wms = params["Wm"].reshape(3, D, D)
    bm = params["bm"].reshape(1, D)

    ones8 = jnp.ones((C, 8), jnp.float32)
    zeros8 = jnp.zeros((C, 8), jnp.float32)
    zerosD = jnp.zeros((C, D), jnp.float32)

    degp = _deg_kernel(dst_r, ones8, zeros8)

    ys = _prep(x_pad, w1s, degp)
    p = _spmm_kernel(ys[0], ys[1], ys[2], src_r, dst_r, zerosD)
    zs = _mid(p, ys, degp, w2s, b1s, g1s, be1s, a1s)
    q = _spmm_kernel(zs[0], zs[1], zs[2], src_r, dst_r, zerosD)
    out = _fin(q, zs, degp, b2s, wms, bm)
    return out[:N]


# speedup vs baseline: 6.7113x; 6.7113x over previous
import functools

import jax
import jax.numpy as jnp
from jax import lax
from jax.experimental import pallas as pl
from jax.experimental.pallas import tpu as pltpu
from jax.experimental.pallas import tpu_sc as plsc

N = 10000
D = 128
E = 320000

NC = 2
NS = 16
NW = NC * NS

NP = 10240
EPT = 10240
EP = EPT * NW
C = 128
NCH = EPT // C
RPS = NP // NS

_mesh = plsc.VectorSubcoreMesh(core_axis_name="c", subcore_axis_name="s")


@functools.partial(
    pl.kernel,
    out_type=jax.ShapeDtypeStruct((3, NC, NP, D), jnp.float32),
    mesh=_mesh,
    scratch_types=[
        pltpu.VMEM_SHARED((NP, D), jnp.float32),
        pltpu.VMEM((C, D), jnp.float32),
        pltpu.VMEM((NCH, C), jnp.int32),
    ],
)
def _deg_kernel(dst_r, ones_h, zeros_h, out, acc, ones_v, didx_v):
    c = lax.axis_index("c")
    s = lax.axis_index("s")
    w = c * NS + s
    for k in range(3):
        pltpu.sync_copy(zeros_h, ones_v)
        for r in range(RPS // C):
            pltpu.sync_copy(ones_v, acc.at[pl.ds(s * RPS + r * C, C)])
        pltpu.sync_copy(ones_h, ones_v)
        plsc.subcore_barrier()
        pltpu.sync_copy(dst_r.at[k, w], didx_v)

        def body(j, carry):
            pltpu.sync_copy(ones_v, acc.at[didx_v.at[j]], add=True)
            return carry

        lax.fori_loop(0, NCH, body, 0)
        plsc.subcore_barrier()
        pltpu.sync_copy(
            acc.at[pl.ds(s * RPS, RPS)], out.at[k, c, pl.ds(s * RPS, RPS)]
        )
        plsc.subcore_barrier()


@functools.partial(
    pl.kernel,
    out_type=jax.ShapeDtypeStruct((3, NC, NP, D), jnp.float32),
    mesh=_mesh,
    scratch_types=[
        pltpu.VMEM_SHARED((NP, D), jnp.float32),
        pltpu.VMEM((C, D), jnp.float32),
        pltpu.VMEM((NCH, C), jnp.int32),
        pltpu.VMEM((NCH, C), jnp.int32),
        pltpu.SemaphoreType.DMA,
    ],
)
def _spmm_kernel(h0, h1, h2, src_r, dst_r, zeros, out,
                 acc, rbuf, sidx_v, didx_v, sem):
    c = lax.axis_index("c")
    s = lax.axis_index("s")
    w = c * NS + s
    for k, h in enumerate((h0, h1, h2)):
        pltpu.sync_copy(zeros, rbuf)
        for r in range(RPS // C):
            pltpu.sync_copy(rbuf, acc.at[pl.ds(s * RPS + r * C, C)])
        plsc.subcore_barrier()
        pltpu.sync_copy(src_r.at[k, w], sidx_v)
        pltpu.sync_copy(dst_r.at[k, w], didx_v)

        def body(j, carry):
            pltpu.async_copy(h.at[sidx_v.at[j]], rbuf, sem).wait()
            pltpu.sync_copy(rbuf, acc.at[didx_v.at[j]], add=True)
            return carry

        lax.fori_loop(0, NCH, body, 0)
        plsc.subcore_barrier()
        pltpu.sync_copy(
            acc.at[pl.ds(s * RPS, RPS)], out.at[k, c, pl.ds(s * RPS, RPS)]
        )
        plsc.subcore_barrier()


_BLK = 1024
_GR = NP // _BLK


def _dot(a, b):
    return jnp.dot(a, b, preferred_element_type=jnp.float32,
                   precision=lax.Precision.HIGHEST)


def _prep_body(x_ref, w1_ref, degp_ref, out_ref):
    deg = degp_ref[0, 0] + degp_ref[0, 1]
    dinv = lax.rsqrt(deg[:, 0:1] + 1.0)
    out_ref[0] = dinv * _dot(x_ref[...], w1_ref[0])


def _prep(x_pad, w1s, degp):
    return pl.pallas_call(
        _prep_body,
        grid=(3, _GR),
        in_specs=[
            pl.BlockSpec((_BLK, D), lambda k, i: (i, 0)),
            pl.BlockSpec((1, D, D), lambda k, i: (k, 0, 0)),
            pl.BlockSpec((1, NC, _BLK, D), lambda k, i: (k, 0, i, 0)),
        ],
        out_specs=pl.BlockSpec((1, _BLK, D), lambda k, i: (k, i, 0)),
        out_shape=jax.ShapeDtypeStruct((3, NP, D), jnp.float32),
    )(x_pad, w1s, degp)


def _mid_body(p_ref, ys_ref, degp_ref, w2_ref, b1_ref, g1_ref, be1_ref,
              a1_ref, out_ref):
    k = pl.program_id(0)
    deg = degp_ref[0, 0] + degp_ref[0, 1]
    dinv = lax.rsqrt(deg[:, 0:1] + 1.0)
    h = dinv * (p_ref[0, 0] + p_ref[0, 1] + ys_ref[0]) + b1_ref[k]
    m = jnp.mean(h, axis=-1, keepdims=True)
    v = jnp.mean(jnp.square(h - m), axis=-1, keepdims=True)
    hn = (h - m) * lax.rsqrt(v + 1e-5) * g1_ref[k] + be1_ref[k]
    hp = jnp.where(hn >= 0, hn, a1_ref[k] * hn)
    out_ref[0] = dinv * _dot(hp, w2_ref[0])


def _mid(p, ys, degp, w2s, b1s, g1s, be1s, a1s):
    vec = pl.BlockSpec((3, D), lambda k, i: (0, 0))
    return pl.pallas_call(
        _mid_body,
        grid=(3, _GR),
        in_specs=[
            pl.BlockSpec((1, NC, _BLK, D), lambda k, i: (k, 0, i, 0)),
            pl.BlockSpec((1, _BLK, D), lambda k, i: (k, i, 0)),
            pl.BlockSpec((1, NC, _BLK, D), lambda k, i: (k, 0, i, 0)),
            pl.BlockSpec((1, D, D), lambda k, i: (k, 0, 0)),
            vec, vec, vec, vec,
        ],
        out_specs=pl.BlockSpec((1, _BLK, D), lambda k, i: (k, i, 0)),
        out_shape=jax.ShapeDtypeStruct((3, NP, D), jnp.float32),
    )(p, ys, degp, w2s, b1s, g1s, be1s, a1s)


def _fin_body(q_ref, zs_ref, degp_ref, b2_ref, wm_ref, bm_ref, out_ref):
    acc = jnp.zeros((_BLK, D), jnp.float32)
    for k in range(3):
        deg = degp_ref[k, 0] + degp_ref[k, 1]
        dinv = lax.rsqrt(deg[:, 0:1] + 1.0)
        hk = dinv * (q_ref[k, 0] + q_ref[k, 1] + zs_ref[k]) + b2_ref[k]
        acc = acc + _dot(hk, wm_ref[k]) + hk
    out_ref[...] = jnp.maximum(acc + bm_ref[0], 0.0)


def _fin(q, zs, degp, b2s, wms, bm):
    return pl.pallas_call(
        _fin_body,
        grid=(_GR,),
        in_specs=[
            pl.BlockSpec((3, NC, _BLK, D), lambda i: (0, 0, i, 0)),
            pl.BlockSpec((3, _BLK, D), lambda i: (0, i, 0)),
            pl.BlockSpec((3, NC, _BLK, D), lambda i: (0, 0, i, 0)),
            pl.BlockSpec((3, D), lambda i: (0, 0)),
            pl.BlockSpec((3, D, D), lambda i: (0, 0, 0)),
            pl.BlockSpec((1, D), lambda i: (0, 0)),
        ],
        out_specs=pl.BlockSpec((_BLK, D), lambda i: (i, 0)),
        out_shape=jax.ShapeDtypeStruct((NP, D), jnp.float32),
    )(q, zs, degp, b2s, wms, bm)


def _prep_edges(adj):
    pad = jnp.full((EP - E,), NP - 1, dtype=jnp.int32)
    src = jnp.concatenate([adj[0].astype(jnp.int32), pad]).reshape(NW, NCH, C)
    dst = jnp.concatenate([adj[1].astype(jnp.int32), pad]).reshape(NW, NCH, C)
    return src, dst


@jax.jit
def kernel(x, params, adj_intra, adj_masked, adj):
    x_pad = jnp.pad(x, ((0, NP - N), (0, 0)))
    edges = [_prep_edges(a) for a in (adj_intra, adj_masked, adj)]
    src_r = jnp.stack([e[0] for e in edges])
    dst_r = jnp.stack([e[1] for e in edges])

    encs = [params["intra"], params["masked"], params["full"]]
    w1s = jnp.stack([p["W1"] for p in encs])
    w2s = jnp.stack([p["W2"] for p in encs])
    b1s = jnp.stack([p["b1"] for p in encs])
    b2s = jnp.stack([p["b2"] for p in encs])
    g1s = jnp.stack([p["g1"] for p in encs])
    be1s = jnp.stack([p["be1"] for p in encs])
    a1s = jnp.stack([jnp.full((D,), p["a1"], jnp.float32) for p in encs])
    wms = params["Wm"].reshape(3, D, D)
    bm = params["bm"].reshape(1, D)

    onesD = jnp.ones((C, D), jnp.float32)
    zerosD = jnp.zeros((C, D), jnp.float32)

    degp = _deg_kernel(dst_r, onesD, zerosD)

    ys = _prep(x_pad, w1s, degp)
    p = _spmm_kernel(ys[0], ys[1], ys[2], src_r, dst_r, zerosD)
    zs = _mid(p, ys, degp, w2s, b1s, g1s, be1s, a1s)
    q = _spmm_kernel(zs[0], zs[1], zs[2], src_r, dst_r, zerosD)
    out = _fin(q, zs, degp, b2s, wms, bm)
    return out[:N]


# speedup vs baseline: 22.9513x; 3.4198x over previous
import functools

import jax
import jax.numpy as jnp
from jax import lax
from jax.experimental import pallas as pl
from jax.experimental.pallas import tpu as pltpu
from jax.experimental.pallas import tpu_sc as plsc

N = 10000
D = 128
E = 320000

NC = 2
NS = 16
NW = NC * NS

NP = 10240
EPT = 10240
EP = EPT * NW
C = 128
NCH = EPT // C
NCH2 = NCH // 2
RPS = NP // NS

_mesh = plsc.VectorSubcoreMesh(core_axis_name="c", subcore_axis_name="s")


@functools.partial(
    pl.kernel,
    out_type=jax.ShapeDtypeStruct((3, NC, NP, D), jnp.float32),
    mesh=_mesh,
    scratch_types=[
        pltpu.VMEM_SHARED((NP, D), jnp.float32),
        pltpu.VMEM((C, D), jnp.float32),
        pltpu.VMEM((2, NCH2, C), jnp.int32),
    ],
)
def _deg_kernel(dst_r, ones_h, zeros_h, out, acc, ones_v, didx_v):
    c = lax.axis_index("c")
    s = lax.axis_index("s")
    w = c * NS + s
    for k in range(3):
        pltpu.sync_copy(zeros_h, ones_v)
        for r in range(RPS // C):
            pltpu.sync_copy(ones_v, acc.at[pl.ds(s * RPS + r * C, C)])
        pltpu.sync_copy(ones_h, ones_v)
        plsc.subcore_barrier()
        pltpu.sync_copy(dst_r.at[k, w], didx_v)

        for half in range(2):
            def body(j, carry, half=half):
                pltpu.sync_copy(ones_v, acc.at[didx_v.at[half, j]], add=True)
                return carry

            lax.fori_loop(0, NCH2, body, 0)
        plsc.subcore_barrier()
        pltpu.sync_copy(
            acc.at[pl.ds(s * RPS, RPS)], out.at[k, c, pl.ds(s * RPS, RPS)]
        )
        plsc.subcore_barrier()


@functools.partial(
    pl.kernel,
    out_type=jax.ShapeDtypeStruct((3, NC, NP, D), jnp.float32),
    mesh=_mesh,
    scratch_types=[
        pltpu.VMEM_SHARED((NP, D), jnp.float32),
        pltpu.VMEM((C, D), jnp.float32),
        pltpu.VMEM((C, D), jnp.float32),
        pltpu.VMEM((NCH2, C), jnp.int32),
        pltpu.VMEM((NCH2, C), jnp.int32),
        pltpu.SemaphoreType.DMA,
        pltpu.SemaphoreType.DMA,
    ],
)
def _spmm_kernel(h0, h1, h2, src_r, dst_r, zeros, out,
                 acc, rbufa, rbufb, sidx_v, didx_v, sema, semb):
    c = lax.axis_index("c")
    s = lax.axis_index("s")
    w = c * NS + s
    for k, h in enumerate((h0, h1, h2)):
        pltpu.sync_copy(zeros, rbufa)
        for r in range(RPS // C):
            pltpu.sync_copy(rbufa, acc.at[pl.ds(s * RPS + r * C, C)])
        plsc.subcore_barrier()
        for half in range(2):
            pltpu.sync_copy(src_r.at[k, w, half], sidx_v)
            pltpu.sync_copy(dst_r.at[k, w, half], didx_v)

            pltpu.async_copy(h.at[sidx_v.at[0]], rbufa, sema)
            pltpu.async_copy(h.at[sidx_v.at[1]], rbufb, semb)

            def body(j2, carry):
                j = 2 * j2
                pltpu.make_async_copy(h.at[sidx_v.at[j]], rbufa, sema).wait()
                pltpu.sync_copy(rbufa, acc.at[didx_v.at[j]], add=True)

                @pl.when(j + 2 < NCH2)
                def _():
                    pltpu.async_copy(h.at[sidx_v.at[j + 2]], rbufa, sema)

                pltpu.make_async_copy(
                    h.at[sidx_v.at[j + 1]], rbufb, semb).wait()
                pltpu.sync_copy(rbufb, acc.at[didx_v.at[j + 1]], add=True)

                @pl.when(j + 3 < NCH2)
                def _():
                    pltpu.async_copy(h.at[sidx_v.at[j + 3]], rbufb, semb)

                return carry

            lax.fori_loop(0, NCH2 // 2, body, 0)
        plsc.subcore_barrier()
        pltpu.sync_copy(
            acc.at[pl.ds(s * RPS, RPS)], out.at[k, c, pl.ds(s * RPS, RPS)]
        )
        plsc.subcore_barrier()


_BLK = 1024
_GR = NP // _BLK


def _dot(a, b):
    return jnp.dot(a, b, preferred_element_type=jnp.float32,
                   precision=lax.Precision.HIGHEST)


def _prep_body(x_ref, w1_ref, degp_ref, out_ref):
    deg = degp_ref[0, 0] + degp_ref[0, 1]
    dinv = lax.rsqrt(deg[:, 0:1] + 1.0)
    out_ref[0] = dinv * _dot(x_ref[...], w1_ref[0])


def _prep(x_pad, w1s, degp):
    return pl.pallas_call(
        _prep_body,
        grid=(3, _GR),
        in_specs=[
            pl.BlockSpec((_BLK, D), lambda k, i: (i, 0)),
            pl.BlockSpec((1, D, D), lambda k, i: (k, 0, 0)),
            pl.BlockSpec((1, NC, _BLK, D), lambda k, i: (k, 0, i, 0)),
        ],
        out_specs=pl.BlockSpec((1, _BLK, D), lambda k, i: (k, i, 0)),
        out_shape=jax.ShapeDtypeStruct((3, NP, D), jnp.float32),
    )(x_pad, w1s, degp)


def _mid_body(p_ref, ys_ref, degp_ref, w2_ref, b1_ref, g1_ref, be1_ref,
              a1_ref, out_ref):
    k = pl.program_id(0)
    deg = degp_ref[0, 0] + degp_ref[0, 1]
    dinv = lax.rsqrt(deg[:, 0:1] + 1.0)
    h = dinv * (p_ref[0, 0] + p_ref[0, 1] + ys_ref[0]) + b1_ref[k]
    m = jnp.mean(h, axis=-1, keepdims=True)
    v = jnp.mean(jnp.square(h - m), axis=-1, keepdims=True)
    hn = (h - m) * lax.rsqrt(v + 1e-5) * g1_ref[k] + be1_ref[k]
    hp = jnp.where(hn >= 0, hn, a1_ref[k] * hn)
    out_ref[0] = dinv * _dot(hp, w2_ref[0])


def _mid(p, ys, degp, w2s, b1s, g1s, be1s, a1s):
    vec = pl.BlockSpec((3, D), lambda k, i: (0, 0))
    return pl.pallas_call(
        _mid_body,
        grid=(3, _GR),
        in_specs=[
            pl.BlockSpec((1, NC, _BLK, D), lambda k, i: (k, 0, i, 0)),
            pl.BlockSpec((1, _BLK, D), lambda k, i: (k, i, 0)),
            pl.BlockSpec((1, NC, _BLK, D), lambda k, i: (k, 0, i, 0)),
            pl.BlockSpec((1, D, D), lambda k, i: (k, 0, 0)),
            vec, vec, vec, vec,
        ],
        out_specs=pl.BlockSpec((1, _BLK, D), lambda k, i: (k, i, 0)),
        out_shape=jax.ShapeDtypeStruct((3, NP, D), jnp.float32),
    )(p, ys, degp, w2s, b1s, g1s, be1s, a1s)


def _fin_body(q_ref, zs_ref, degp_ref, b2_ref, wm_ref, bm_ref, out_ref):
    acc = jnp.zeros((_BLK, D), jnp.float32)
    for k in range(3):
        deg = degp_ref[k, 0] + degp_ref[k, 1]
        dinv = lax.rsqrt(deg[:, 0:1] + 1.0)
        hk = dinv * (q_ref[k, 0] + q_ref[k, 1] + zs_ref[k]) + b2_ref[k]
        acc = acc + _dot(hk, wm_ref[k]) + hk
    out_ref[...] = jnp.maximum(acc + bm_ref[0], 0.0)


def _fin(q, zs, degp, b2s, wms, bm):
    return pl.pallas_call(
        _fin_body,
        grid=(_GR,),
        in_specs=[
            pl.BlockSpec((3, NC, _BLK, D), lambda i: (0, 0, i, 0)),
            pl.BlockSpec((3, _BLK, D), lambda i: (0, i, 0)),
            pl.BlockSpec((3, NC, _BLK, D), lambda i: (0, 0, i, 0)),
            pl.BlockSpec((3, D), lambda i: (0, 0)),
            pl.BlockSpec((3, D, D), lambda i: (0, 0, 0)),
            pl.BlockSpec((1, D), lambda i: (0, 0)),
        ],
        out_specs=pl.BlockSpec((_BLK, D), lambda i: (i, 0)),
        out_shape=jax.ShapeDtypeStruct((NP, D), jnp.float32),
    )(q, zs, degp, b2s, wms, bm)


def _prep_edges(adj):
    pad = N + jnp.arange(EP - E, dtype=jnp.int32) % (NP - N)
    src = jnp.concatenate([adj[0].astype(jnp.int32), pad])
    dst = jnp.concatenate([adj[1].astype(jnp.int32), pad])
    return (src.reshape(NW, 2, NCH2, C), dst.reshape(NW, 2, NCH2, C))


@jax.jit
def kernel(x, params, adj_intra, adj_masked, adj):
    x_pad = jnp.pad(x, ((0, NP - N), (0, 0)))
    edges = [_prep_edges(a) for a in (adj_intra, adj_masked, adj)]
    src_r = jnp.stack([e[0] for e in edges])
    dst_r = jnp.stack([e[1] for e in edges])

    encs = [params["intra"], params["masked"], params["full"]]
    w1s = jnp.stack([p["W1"] for p in encs])
    w2s = jnp.stack([p["W2"] for p in encs])
    b1s = jnp.stack([p["b1"] for p in encs])
    b2s = jnp.stack([p["b2"] for p in encs])
    g1s = jnp.stack([p["g1"] for p in encs])
    be1s = jnp.stack([p["be1"] for p in encs])
    a1s = jnp.stack([jnp.full((D,), p["a1"], jnp.float32) for p in encs])
    wms = params["Wm"].reshape(3, D, D)
    bm = params["bm"].reshape(1, D)

    onesD = jnp.ones((C, D), jnp.float32)
    zerosD = jnp.zeros((C, D), jnp.float32)

    degp = _deg_kernel(dst_r, onesD, zerosD)

    ys = _prep(x_pad, w1s, degp)
    p = _spmm_kernel(ys[0], ys[1], ys[2], src_r, dst_r, zerosD)
    zs = _mid(p, ys, degp, w2s, b1s, g1s, be1s, a1s)
    q = _spmm_kernel(zs[0], zs[1], zs[2], src_r, dst_r, zerosD)
    out = _fin(q, zs, degp, b2s, wms, bm)
    return out[:N]


# speedup vs baseline: 23.0119x; 1.0026x over previous
import functools

import jax
import jax.numpy as jnp
from jax import lax
from jax.experimental import pallas as pl
from jax.experimental.pallas import tpu as pltpu
from jax.experimental.pallas import tpu_sc as plsc

N = 10000
D = 128
E = 320000

NC = 2
NS = 16
NW = NC * NS

NP = 10240
EPT = 10240
EP = EPT * NW
C = 128
NCH = EPT // C
NCH2 = NCH // 2
RPS = NP // NS

_mesh = plsc.VectorSubcoreMesh(core_axis_name="c", subcore_axis_name="s")


@functools.partial(
    pl.kernel,
    out_type=jax.ShapeDtypeStruct((3, NC, NP, D), jnp.float32),
    mesh=_mesh,
    scratch_types=[
        pltpu.VMEM_SHARED((NP, D), jnp.float32),
        pltpu.VMEM((C, D), jnp.float32),
        pltpu.VMEM((2, NCH2, C), jnp.int32),
    ],
)
def _deg_kernel(dst_r, ones_h, zeros_h, out, acc, ones_v, didx_v):
    c = lax.axis_index("c")
    s = lax.axis_index("s")
    w = c * NS + s
    for k in range(3):
        pltpu.sync_copy(zeros_h, ones_v)
        for r in range(RPS // C):
            pltpu.sync_copy(ones_v, acc.at[pl.ds(s * RPS + r * C, C)])
        pltpu.sync_copy(ones_h, ones_v)
        plsc.subcore_barrier()
        pltpu.sync_copy(dst_r.at[k, w], didx_v)

        for half in range(2):
            def body(j, carry, half=half):
                pltpu.sync_copy(ones_v, acc.at[didx_v.at[half, j]], add=True)
                return carry

            lax.fori_loop(0, NCH2, body, 0)
        plsc.subcore_barrier()
        pltpu.sync_copy(
            acc.at[pl.ds(s * RPS, RPS)], out.at[k, c, pl.ds(s * RPS, RPS)]
        )
        plsc.subcore_barrier()


@functools.partial(
    pl.kernel,
    out_type=jax.ShapeDtypeStruct((3, NC, NP, D), jnp.float32),
    mesh=_mesh,
    scratch_types=[
        pltpu.VMEM_SHARED((NP, D), jnp.float32),
        pltpu.VMEM((C, D), jnp.float32),
        pltpu.VMEM((C, D), jnp.float32),
        pltpu.VMEM((NCH2, C), jnp.int32),
        pltpu.VMEM((NCH2, C), jnp.int32),
        pltpu.SemaphoreType.DMA,
        pltpu.SemaphoreType.DMA,
    ],
)
def _spmm_kernel(h0, h1, h2, src_r, dst_r, zeros, out,
                 acc, rbufa, rbufb, sidx_v, didx_v, sema, semb):
    c = lax.axis_index("c")
    s = lax.axis_index("s")
    w = c * NS + s
    for k, h in enumerate((h0, h1, h2)):
        pltpu.sync_copy(zeros, rbufa)
        for r in range(RPS // C):
            pltpu.sync_copy(rbufa, acc.at[pl.ds(s * RPS + r * C, C)])
        plsc.subcore_barrier()
        for half in range(2):
            pltpu.sync_copy(src_r.at[k, w, half], sidx_v)
            pltpu.sync_copy(dst_r.at[k, w, half], didx_v)

            pltpu.async_copy(h.at[sidx_v.at[0]], rbufa, sema)
            pltpu.async_copy(h.at[sidx_v.at[1]], rbufb, semb)

            def body(j2, carry):
                j = 2 * j2
                pltpu.make_async_copy(h.at[sidx_v.at[j]], rbufa, sema).wait()
                pltpu.sync_copy(rbufa, acc.at[didx_v.at[j]], add=True)

                @pl.when(j + 2 < NCH2)
                def _():
                    pltpu.async_copy(h.at[sidx_v.at[j + 2]], rbufa, sema)

                pltpu.make_async_copy(
                    h.at[sidx_v.at[j + 1]], rbufb, semb).wait()
                pltpu.sync_copy(rbufb, acc.at[didx_v.at[j + 1]], add=True)

                @pl.when(j + 3 < NCH2)
                def _():
                    pltpu.async_copy(h.at[sidx_v.at[j + 3]], rbufb, semb)

                return carry

            lax.fori_loop(0, NCH2 // 2, body, 0)
        plsc.subcore_barrier()
        pltpu.sync_copy(
            acc.at[pl.ds(s * RPS, RPS)], out.at[k, c, pl.ds(s * RPS, RPS)]
        )
        plsc.subcore_barrier()


_BLK = 1024
_GR = NP // _BLK


def _dot(a, b):
    return jnp.dot(a, b, preferred_element_type=jnp.float32,
                   precision=lax.Precision.HIGHEST)


def _mm_body(x_ref, w1_ref, out_ref):
    out_ref[0] = _dot(x_ref[...], w1_ref[0])


def _mm(x_pad, w1s):
    return pl.pallas_call(
        _mm_body,
        grid=(3, _GR),
        in_specs=[
            pl.BlockSpec((_BLK, D), lambda k, i: (i, 0)),
            pl.BlockSpec((1, D, D), lambda k, i: (k, 0, 0)),
        ],
        out_specs=pl.BlockSpec((1, _BLK, D), lambda k, i: (k, i, 0)),
        out_shape=jax.ShapeDtypeStruct((3, NP, D), jnp.float32),
    )(x_pad, w1s)


def _scale_body(r_ref, degp_ref, out_ref):
    deg = degp_ref[0, 0, :, 0:1] + degp_ref[0, 1, :, 0:1]
    out_ref[0] = lax.rsqrt(deg + 1.0) * r_ref[0]


def _scale(r, degp):
    return pl.pallas_call(
        _scale_body,
        grid=(3, _GR),
        in_specs=[
            pl.BlockSpec((1, _BLK, D), lambda k, i: (k, i, 0)),
            pl.BlockSpec((1, NC, _BLK, D), lambda k, i: (k, 0, i, 0)),
        ],
        out_specs=pl.BlockSpec((1, _BLK, D), lambda k, i: (k, i, 0)),
        out_shape=jax.ShapeDtypeStruct((3, NP, D), jnp.float32),
    )(r, degp)


def _mid_body(p_ref, ys_ref, degp_ref, w2_ref, b1_ref, g1_ref, be1_ref,
              a1_ref, out_ref):
    k = pl.program_id(0)
    deg = degp_ref[0, 0, :, 0:1] + degp_ref[0, 1, :, 0:1]
    dinv = lax.rsqrt(deg + 1.0)
    h = dinv * (p_ref[0, 0] + p_ref[0, 1] + ys_ref[0]) + b1_ref[k]
    m = jnp.mean(h, axis=-1, keepdims=True)
    v = jnp.mean(jnp.square(h - m), axis=-1, keepdims=True)
    hn = (h - m) * lax.rsqrt(v + 1e-5) * g1_ref[k] + be1_ref[k]
    hp = jnp.where(hn >= 0, hn, a1_ref[k] * hn)
    out_ref[0] = dinv * _dot(hp, w2_ref[0])


def _mid(p, ys, degp, w2s, b1s, g1s, be1s, a1s):
    vec = pl.BlockSpec((3, D), lambda k, i: (0, 0))
    return pl.pallas_call(
        _mid_body,
        grid=(3, _GR),
        in_specs=[
            pl.BlockSpec((1, NC, _BLK, D), lambda k, i: (k, 0, i, 0)),
            pl.BlockSpec((1, _BLK, D), lambda k, i: (k, i, 0)),
            pl.BlockSpec((1, NC, _BLK, D), lambda k, i: (k, 0, i, 0)),
            pl.BlockSpec((1, D, D), lambda k, i: (k, 0, 0)),
            vec, vec, vec, vec,
        ],
        out_specs=pl.BlockSpec((1, _BLK, D), lambda k, i: (k, i, 0)),
        out_shape=jax.ShapeDtypeStruct((3, NP, D), jnp.float32),
    )(p, ys, degp, w2s, b1s, g1s, be1s, a1s)


def _fin_body(q_ref, zs_ref, degp_ref, b2_ref, wm_ref, bm_ref, out_ref):
    acc = jnp.zeros((_BLK, D), jnp.float32)
    for k in range(3):
        deg = degp_ref[k, 0, :, 0:1] + degp_ref[k, 1, :, 0:1]
        dinv = lax.rsqrt(deg + 1.0)
        hk = dinv * (q_ref[k, 0] + q_ref[k, 1] + zs_ref[k]) + b2_ref[k]
        acc = acc + _dot(hk, wm_ref[k]) + hk
    out_ref[...] = jnp.maximum(acc + bm_ref[0], 0.0)


def _fin(q, zs, degp, b2s, wms, bm):
    return pl.pallas_call(
        _fin_body,
        grid=(_GR,),
        in_specs=[
            pl.BlockSpec((3, NC, _BLK, D), lambda i: (0, 0, i, 0)),
            pl.BlockSpec((3, _BLK, D), lambda i: (0, i, 0)),
            pl.BlockSpec((3, NC, _BLK, D), lambda i: (0, 0, i, 0)),
            pl.BlockSpec((3, D), lambda i: (0, 0)),
            pl.BlockSpec((3, D, D), lambda i: (0, 0, 0)),
            pl.BlockSpec((1, D), lambda i: (0, 0)),
        ],
        out_specs=pl.BlockSpec((_BLK, D), lambda i: (i, 0)),
        out_shape=jax.ShapeDtypeStruct((NP, D), jnp.float32),
    )(q, zs, degp, b2s, wms, bm)


def _prep_edges(adj):
    pad = N + jnp.arange(EP - E, dtype=jnp.int32) % (NP - N)
    src = jnp.concatenate([adj[0].astype(jnp.int32), pad])
    dst = jnp.concatenate([adj[1].astype(jnp.int32), pad])
    return (src.reshape(NW, 2, NCH2, C), dst.reshape(NW, 2, NCH2, C))


@jax.jit
def kernel(x, params, adj_intra, adj_masked, adj):
    x_pad = jnp.pad(x, ((0, NP - N), (0, 0)))
    edges = [_prep_edges(a) for a in (adj_intra, adj_masked, adj)]
    src_r = jnp.stack([e[0] for e in edges])
    dst_r = jnp.stack([e[1] for e in edges])

    encs = [params["intra"], params["masked"], params["full"]]
    w1s = jnp.stack([p["W1"] for p in encs])
    w2s = jnp.stack([p["W2"] for p in encs])
    b1s = jnp.stack([p["b1"] for p in encs])
    b2s = jnp.stack([p["b2"] for p in encs])
    g1s = jnp.stack([p["g1"] for p in encs])
    be1s = jnp.stack([p["be1"] for p in encs])
    a1s = jnp.stack([jnp.full((D,), p["a1"], jnp.float32) for p in encs])
    wms = params["Wm"].reshape(3, D, D)
    bm = params["bm"].reshape(1, D)

    onesD = jnp.ones((C, D), jnp.float32)
    zerosD = jnp.zeros((C, D), jnp.float32)

    degp = _deg_kernel(dst_r, onesD, zerosD)

    ys = _scale(_mm(x_pad, w1s), degp)
    p = _spmm_kernel(ys[0], ys[1], ys[2], src_r, dst_r, zerosD)
    zs = _mid(p, ys, degp, w2s, b1s, g1s, be1s, a1s)
    q = _spmm_kernel(zs[0], zs[1], zs[2], src_r, dst_r, zerosD)
    out = _fin(q, zs, degp, b2s, wms, bm)
    return out[:N]


# speedup vs baseline: 26.7994x; 1.1646x over previous
import functools

import jax
import jax.numpy as jnp
from jax import lax
from jax.experimental import pallas as pl
from jax.experimental.pallas import tpu as pltpu
from jax.experimental.pallas import tpu_sc as plsc

N = 10000
D = 128
E = 320000

NC = 2
NS = 16
NW = NC * NS

NP = 10240
EPT = 10240
EP = EPT * NW
C = 128
NCH = EPT // C
NCH2 = NCH // 2
RPS = NP // NS

_mesh = plsc.VectorSubcoreMesh(core_axis_name="c", subcore_axis_name="s")


@functools.partial(
    pl.kernel,
    out_type=jax.ShapeDtypeStruct((3, NC, NP, 16), jnp.float32),
    mesh=_mesh,
    scratch_types=[
        pltpu.VMEM_SHARED((NP, 16), jnp.float32),
        pltpu.VMEM((C, 16), jnp.float32),
        pltpu.VMEM((RPS, 16), jnp.float32),
        pltpu.VMEM((2, NCH2, C), jnp.int32),
    ],
)
def _deg_kernel(dst_r, out, acc, ones_v, zer_v, didx_v):
    c = lax.axis_index("c")
    s = lax.axis_index("s")
    w = c * NS + s
    for i in range(C):
        ones_v[i, :] = jnp.ones((16,), jnp.float32)

    def zero(i, carry):
        zer_v[i, :] = jnp.zeros((16,), jnp.float32)
        return carry

    lax.fori_loop(0, RPS, zero, 0)
    for k in range(3):
        pltpu.sync_copy(zer_v, acc.at[pl.ds(s * RPS, RPS)])
        plsc.subcore_barrier()
        pltpu.sync_copy(dst_r.at[k, w], didx_v)

        for half in range(2):
            def body(j, carry, half=half):
                pltpu.sync_copy(ones_v, acc.at[didx_v.at[half, j]], add=True)
                return carry

            lax.fori_loop(0, NCH2, body, 0)
        plsc.subcore_barrier()
        pltpu.sync_copy(
            acc.at[pl.ds(s * RPS, RPS)], out.at[k, c, pl.ds(s * RPS, RPS)]
        )
        plsc.subcore_barrier()


@functools.partial(
    pl.kernel,
    out_type=jax.ShapeDtypeStruct((3, NC, NP, D), jnp.float32),
    mesh=_mesh,
    scratch_types=[
        pltpu.VMEM_SHARED((NP, D), jnp.float32),
        pltpu.VMEM((C, D), jnp.float32),
        pltpu.VMEM((C, D), jnp.float32),
        pltpu.VMEM((NCH2, C), jnp.int32),
        pltpu.VMEM((NCH2, C), jnp.int32),
        pltpu.SemaphoreType.DMA,
        pltpu.SemaphoreType.DMA,
    ],
)
def _spmm_kernel(h0, h1, h2, src_r, dst_r, zeros, out,
                 acc, rbufa, rbufb, sidx_v, didx_v, sema, semb):
    c = lax.axis_index("c")
    s = lax.axis_index("s")
    w = c * NS + s
    for k, h in enumerate((h0, h1, h2)):
        pltpu.sync_copy(zeros, rbufa)
        for r in range(RPS // C):
            pltpu.sync_copy(rbufa, acc.at[pl.ds(s * RPS + r * C, C)])
        plsc.subcore_barrier()
        for half in range(2):
            pltpu.sync_copy(src_r.at[k, w, half], sidx_v)
            pltpu.sync_copy(dst_r.at[k, w, half], didx_v)

            pltpu.async_copy(h.at[sidx_v.at[0]], rbufa, sema)
            pltpu.async_copy(h.at[sidx_v.at[1]], rbufb, semb)

            def body(j2, carry):
                j = 2 * j2
                pltpu.make_async_copy(h.at[sidx_v.at[j]], rbufa, sema).wait()
                pltpu.sync_copy(rbufa, acc.at[didx_v.at[j]], add=True)

                @pl.when(j + 2 < NCH2)
                def _():
                    pltpu.async_copy(h.at[sidx_v.at[j + 2]], rbufa, sema)

                pltpu.make_async_copy(
                    h.at[sidx_v.at[j + 1]], rbufb, semb).wait()
                pltpu.sync_copy(rbufb, acc.at[didx_v.at[j + 1]], add=True)

                @pl.when(j + 3 < NCH2)
                def _():
                    pltpu.async_copy(h.at[sidx_v.at[j + 3]], rbufb, semb)

                return carry

            lax.fori_loop(0, NCH2 // 2, body, 0)
        plsc.subcore_barrier()
        pltpu.sync_copy(
            acc.at[pl.ds(s * RPS, RPS)], out.at[k, c, pl.ds(s * RPS, RPS)]
        )
        plsc.subcore_barrier()


_BLK = 1024
_GR = NP // _BLK


def _dot(a, b):
    return jnp.dot(a, b, preferred_element_type=jnp.float32,
                   precision=lax.Precision.HIGHEST)


def _mm_body(x_ref, w1_ref, out_ref):
    out_ref[0] = _dot(x_ref[...], w1_ref[0])


def _mm(x_pad, w1s):
    return pl.pallas_call(
        _mm_body,
        grid=(3, _GR),
        in_specs=[
            pl.BlockSpec((_BLK, D), lambda k, i: (i, 0)),
            pl.BlockSpec((1, D, D), lambda k, i: (k, 0, 0)),
        ],
        out_specs=pl.BlockSpec((1, _BLK, D), lambda k, i: (k, i, 0)),
        out_shape=jax.ShapeDtypeStruct((3, NP, D), jnp.float32),
    )(x_pad, w1s)


def _scale_body(r_ref, degp_ref, out_ref):
    deg = degp_ref[0, 0, :, 0:1] + degp_ref[0, 1, :, 0:1]
    out_ref[0] = lax.rsqrt(deg + 1.0) * r_ref[0]


def _scale(r, degp):
    return pl.pallas_call(
        _scale_body,
        grid=(3, _GR),
        in_specs=[
            pl.BlockSpec((1, _BLK, D), lambda k, i: (k, i, 0)),
            pl.BlockSpec((1, NC, _BLK, 16), lambda k, i: (k, 0, i, 0)),
        ],
        out_specs=pl.BlockSpec((1, _BLK, D), lambda k, i: (k, i, 0)),
        out_shape=jax.ShapeDtypeStruct((3, NP, D), jnp.float32),
    )(r, degp)


def _mid_body(p_ref, ys_ref, degp_ref, w2_ref, b1_ref, g1_ref, be1_ref,
              a1_ref, out_ref):
    k = pl.program_id(0)
    deg = degp_ref[0, 0, :, 0:1] + degp_ref[0, 1, :, 0:1]
    dinv = lax.rsqrt(deg + 1.0)
    h = dinv * (p_ref[0, 0] + p_ref[0, 1] + ys_ref[0]) + b1_ref[k]
    m = jnp.mean(h, axis=-1, keepdims=True)
    v = jnp.mean(jnp.square(h - m), axis=-1, keepdims=True)
    hn = (h - m) * lax.rsqrt(v + 1e-5) * g1_ref[k] + be1_ref[k]
    hp = jnp.where(hn >= 0, hn, a1_ref[k] * hn)
    out_ref[0] = dinv * _dot(hp, w2_ref[0])


def _mid(p, ys, degp, w2s, b1s, g1s, be1s, a1s):
    vec = pl.BlockSpec((3, D), lambda k, i: (0, 0))
    return pl.pallas_call(
        _mid_body,
        grid=(3, _GR),
        in_specs=[
            pl.BlockSpec((1, NC, _BLK, D), lambda k, i: (k, 0, i, 0)),
            pl.BlockSpec((1, _BLK, D), lambda k, i: (k, i, 0)),
            pl.BlockSpec((1, NC, _BLK, 16), lambda k, i: (k, 0, i, 0)),
            pl.BlockSpec((1, D, D), lambda k, i: (k, 0, 0)),
            vec, vec, vec, vec,
        ],
        out_specs=pl.BlockSpec((1, _BLK, D), lambda k, i: (k, i, 0)),
        out_shape=jax.ShapeDtypeStruct((3, NP, D), jnp.float32),
    )(p, ys, degp, w2s, b1s, g1s, be1s, a1s)


def _fin_body(q_ref, zs_ref, degp_ref, b2_ref, wm_ref, bm_ref, out_ref):
    acc = jnp.zeros((_BLK, D), jnp.float32)
    for k in range(3):
        deg = degp_ref[k, 0, :, 0:1] + degp_ref[k, 1, :, 0:1]
        dinv = lax.rsqrt(deg + 1.0)
        hk = dinv * (q_ref[k, 0] + q_ref[k, 1] + zs_ref[k]) + b2_ref[k]
        acc = acc + _dot(hk, wm_ref[k]) + hk
    out_ref[...] = jnp.maximum(acc + bm_ref[0], 0.0)


def _fin(q, zs, degp, b2s, wms, bm):
    return pl.pallas_call(
        _fin_body,
        grid=(_GR,),
        in_specs=[
            pl.BlockSpec((3, NC, _BLK, D), lambda i: (0, 0, i, 0)),
            pl.BlockSpec((3, _BLK, D), lambda i: (0, i, 0)),
            pl.BlockSpec((3, NC, _BLK, 16), lambda i: (0, 0, i, 0)),
            pl.BlockSpec((3, D), lambda i: (0, 0)),
            pl.BlockSpec((3, D, D), lambda i: (0, 0, 0)),
            pl.BlockSpec((1, D), lambda i: (0, 0)),
        ],
        out_specs=pl.BlockSpec((_BLK, D), lambda i: (i, 0)),
        out_shape=jax.ShapeDtypeStruct((NP, D), jnp.float32),
    )(q, zs, degp, b2s, wms, bm)


def _prep_edges(adj):
    pad = N + jnp.arange(EP - E, dtype=jnp.int32) % (NP - N)
    src = jnp.concatenate([adj[0].astype(jnp.int32), pad])
    dst = jnp.concatenate([adj[1].astype(jnp.int32), pad])
    return (src.reshape(NW, 2, NCH2, C), dst.reshape(NW, 2, NCH2, C))


@jax.jit
def kernel(x, params, adj_intra, adj_masked, adj):
    x_pad = jnp.pad(x, ((0, NP - N), (0, 0)))
    edges = [_prep_edges(a) for a in (adj_intra, adj_masked, adj)]
    src_r = jnp.stack([e[0] for e in edges])
    dst_r = jnp.stack([e[1] for e in edges])

    encs = [params["intra"], params["masked"], params["full"]]
    w1s = jnp.stack([p["W1"] for p in encs])
    w2s = jnp.stack([p["W2"] for p in encs])
    b1s = jnp.stack([p["b1"] for p in encs])
    b2s = jnp.stack([p["b2"] for p in encs])
    g1s = jnp.stack([p["g1"] for p in encs])
    be1s = jnp.stack([p["be1"] for p in encs])
    a1s = jnp.stack([jnp.full((D,), p["a1"], jnp.float32) for p in encs])
    wms = params["Wm"].reshape(3, D, D)
    bm = params["bm"].reshape(1, D)

    zerosD = jnp.zeros((C, D), jnp.float32)

    degp = _deg_kernel(dst_r)

    ys = _scale(_mm(x_pad, w1s), degp)
    p = _spmm_kernel(ys[0], ys[1], ys[2], src_r, dst_r, zerosD)
    zs = _mid(p, ys, degp, w2s, b1s, g1s, be1s, a1s)
    q = _spmm_kernel(zs[0], zs[1], zs[2], src_r, dst_r, zerosD)
    out = _fin(q, zs, degp, b2s, wms, bm)
    return out[:N]
